# Initial kernel scaffold; baseline (speedup 1.0000x reference)
import jax
import jax.numpy as jnp
from jax import lax
from jax.experimental import pallas as pl
from jax.experimental.pallas import tpu as pltpu
from jax.experimental.pallas import tpu_sc as plsc

N, E, D, H, C = 10000, 320000, 128, 32, 5
W1AUG = 48
W2AUG = 16
NC, NS = 2, 16
NW = NC * NS
EPW = E // NW
CH = 80
NCHUNK = EPW // CH
ROWS_PT = N // NS


def _tc_a_body(x_ref, wrel_ref, wroot_ref, brel_ref, broot_ref, y1_ref, z1_ref):
    x = x_ref[...]
    y = lax.dot_general(x, wrel_ref[...], (((1,), (1,)), ((), ())),
                        preferred_element_type=jnp.float32)
    ones = jnp.ones((N, 1), jnp.float32)
    pad = jnp.zeros((N, W1AUG - H - 1), jnp.float32)
    y1_ref[...] = jnp.concatenate([y, ones, pad], axis=1)
    z = lax.dot_general(x, wroot_ref[...], (((1,), (1,)), ((), ())),
                        preferred_element_type=jnp.float32)
    z1_ref[...] = z + brel_ref[...] + broot_ref[...]


_tc_a = pl.pallas_call(
    _tc_a_body,
    out_shape=(jax.ShapeDtypeStruct((N, W1AUG), jnp.float32),
               jax.ShapeDtypeStruct((N, H), jnp.float32)),
)


def _tc_b_body(aggp_ref, z1_ref, wrel_ref, wroot_ref, brel_ref, broot_ref,
               y2_ref, aux_ref):
    a = aggp_ref[0] + aggp_ref[1]
    cnt = a[:, H:H + 1]
    out1 = a[:, :H] / jnp.maximum(cnt, 1.0) + z1_ref[...]
    nrm = jnp.sqrt(jnp.sum(out1 * out1, axis=1, keepdims=True))
    h = jnp.maximum(out1 / jnp.maximum(nrm, 1e-12), 0.0)
    y2 = lax.dot_general(h, wrel_ref[...], (((1,), (1,)), ((), ())),
                         preferred_element_type=jnp.float32)
    y2_ref[...] = jnp.concatenate(
        [y2, jnp.zeros((N, W2AUG - C), jnp.float32)], axis=1)
    z2 = lax.dot_general(h, wroot_ref[...], (((1,), (1,)), ((), ())),
                         preferred_element_type=jnp.float32)
    z2 = z2 + brel_ref[...] + broot_ref[...]
    aux_ref[...] = jnp.concatenate(
        [z2, cnt, jnp.zeros((N, W2AUG - C - 1), jnp.float32)], axis=1)


_tc_b = pl.pallas_call(
    _tc_b_body,
    out_shape=(jax.ShapeDtypeStruct((N, W2AUG), jnp.float32),
               jax.ShapeDtypeStruct((N, W2AUG), jnp.float32)),
)


def _tc_c_body(agg2p_ref, aux_ref, out_ref):
    a = agg2p_ref[0] + agg2p_ref[1]
    aux = aux_ref[...]
    cnt = aux[:, C:C + 1]
    o = a[:, :C] / jnp.maximum(cnt, 1.0) + aux[:, :C]
    nrm = jnp.sqrt(jnp.sum(o * o, axis=1, keepdims=True))
    o = o / jnp.maximum(nrm, 1e-12)
    m = jnp.max(o, axis=1, keepdims=True)
    ls = (o - m) - jnp.log(jnp.sum(jnp.exp(o - m), axis=1, keepdims=True))
    out_ref[...] = jnp.concatenate(
        [ls, jnp.zeros((N, W2AUG - C), jnp.float32)], axis=1)


_tc_c = pl.pallas_call(
    _tc_c_body,
    out_shape=jax.ShapeDtypeStruct((N, W2AUG), jnp.float32),
)


_SC_MESH = plsc.VectorSubcoreMesh(core_axis_name="c", subcore_axis_name="s")


def _sc1_body(y1_hbm, src_hbm, dst_hbm, w_hbm, zero_hbm, out_hbm,
              src_v, dst_v, w_v, rows_v, shared):
    c = lax.axis_index("c")
    s = lax.axis_index("s")
    wid = s * NC + c
    pltpu.sync_copy(src_hbm.at[wid], src_v)
    pltpu.sync_copy(dst_hbm.at[wid], dst_v)
    pltpu.sync_copy(w_hbm.at[wid], w_v)
    pltpu.sync_copy(zero_hbm.at[pl.ds(s * ROWS_PT, ROWS_PT)],
                    shared.at[pl.ds(s * ROWS_PT, ROWS_PT)])
    plsc.subcore_barrier()

    def chunk_body(j, carry):
        pltpu.sync_copy(y1_hbm.at[src_v.at[j]], rows_v)
        jj = jnp.full((16,), j, jnp.int32)
        for e in range(CH):
            wb = plsc.load_gather(w_v, [jj, jnp.full((16,), e, jnp.int32)])
            for cb in range(H // 16):
                sl = (e, pl.ds(cb * 16, 16))
                rows_v[sl] = rows_v[sl] * wb
        pltpu.sync_copy(rows_v, shared.at[dst_v.at[j]], add=True)
        return carry

    lax.fori_loop(0, NCHUNK, chunk_body, 0)
    plsc.subcore_barrier()
    pltpu.sync_copy(shared.at[pl.ds(s * ROWS_PT, ROWS_PT)],
                    out_hbm.at[c, pl.ds(s * ROWS_PT, ROWS_PT)])


_sc1 = pl.kernel(
    _sc1_body,
    out_type=jax.ShapeDtypeStruct((NC, N, W1AUG), jnp.float32),
    mesh=_SC_MESH,
    scratch_types=[
        pltpu.VMEM((NCHUNK, CH), jnp.int32),
        pltpu.VMEM((NCHUNK, CH), jnp.int32),
        pltpu.VMEM((NCHUNK, CH), jnp.float32),
        pltpu.VMEM((CH, W1AUG), jnp.float32),
        pltpu.VMEM_SHARED((N, W1AUG), jnp.float32),
    ],
)


def _sc2_body(y2_hbm, src_hbm, dst_hbm, zero_hbm, out_hbm,
              src_v, dst_v, rows_v, shared):
    c = lax.axis_index("c")
    s = lax.axis_index("s")
    wid = s * NC + c
    pltpu.sync_copy(src_hbm.at[wid], src_v)
    pltpu.sync_copy(dst_hbm.at[wid], dst_v)
    pltpu.sync_copy(zero_hbm.at[pl.ds(s * ROWS_PT, ROWS_PT)],
                    shared.at[pl.ds(s * ROWS_PT, ROWS_PT)])
    plsc.subcore_barrier()

    def chunk_body(j, carry):
        pltpu.sync_copy(y2_hbm.at[src_v.at[j]], rows_v)
        pltpu.sync_copy(rows_v, shared.at[dst_v.at[j]], add=True)
        return carry

    lax.fori_loop(0, NCHUNK, chunk_body, 0)
    plsc.subcore_barrier()
    pltpu.sync_copy(shared.at[pl.ds(s * ROWS_PT, ROWS_PT)],
                    out_hbm.at[c, pl.ds(s * ROWS_PT, ROWS_PT)])


_sc2 = pl.kernel(
    _sc2_body,
    out_type=jax.ShapeDtypeStruct((NC, N, W2AUG), jnp.float32),
    mesh=_SC_MESH,
    scratch_types=[
        pltpu.VMEM((NCHUNK, CH), jnp.int32),
        pltpu.VMEM((NCHUNK, CH), jnp.int32),
        pltpu.VMEM((CH, W2AUG), jnp.float32),
        pltpu.VMEM_SHARED((N, W2AUG), jnp.float32),
    ],
)


def kernel(x, edge_index, weight, W1_rel, b1_rel, W1_root, b1_root,
           W2_rel, b2_rel, W2_root, b2_root):
    src = edge_index[0].reshape(NW, NCHUNK, CH)
    dst = edge_index[1].reshape(NW, NCHUNK, CH)
    w3 = weight.reshape(NW, NCHUNK, CH)
    y1aug, z1 = _tc_a(x, W1_rel, W1_root,
                      b1_rel.reshape(1, H), b1_root.reshape(1, H))
    agg1 = _sc1(y1aug, src, dst, w3, jnp.zeros((N, W1AUG), jnp.float32))
    y2aug, aux = _tc_b(agg1, z1, W2_rel, W2_root,
                       b2_rel.reshape(1, C), b2_root.reshape(1, C))
    agg2 = _sc2(y2aug, src, dst, jnp.zeros((N, W2AUG), jnp.float32))
    out = _tc_c(agg2, aux)
    return out[:, :C]


# speedup vs baseline: 9.9967x; 9.9967x over previous
import jax
import jax.numpy as jnp
from jax import lax
from jax.experimental import pallas as pl
from jax.experimental.pallas import tpu as pltpu
from jax.experimental.pallas import tpu_sc as plsc

N, E, D, H, C = 10000, 320000, 128, 32, 5
W1AUG = 48
W2AUG = 16
NC, NS = 2, 16
NW = NC * NS
EPW = E // NW
CH = 80
NCHUNK = EPW // CH
NP = 10240
ROWS_PT = NP // NS


def _tc_a_body(x_ref, wrel_ref, wroot_ref, brel_ref, broot_ref, y1_ref, z1_ref):
    x = x_ref[...]
    y = lax.dot_general(x, wrel_ref[...], (((1,), (1,)), ((), ())),
                        preferred_element_type=jnp.float32)
    ones = jnp.ones((N, 1), jnp.float32)
    pad = jnp.zeros((N, W1AUG - H - 1), jnp.float32)
    y1_ref[...] = jnp.concatenate([y, ones, pad], axis=1)
    z = lax.dot_general(x, wroot_ref[...], (((1,), (1,)), ((), ())),
                        preferred_element_type=jnp.float32)
    z1_ref[...] = z + brel_ref[...] + broot_ref[...]


_tc_a = pl.pallas_call(
    _tc_a_body,
    out_shape=(jax.ShapeDtypeStruct((N, W1AUG), jnp.float32),
               jax.ShapeDtypeStruct((N, H), jnp.float32)),
)


def _tc_b_body(aggp_ref, z1_ref, wrel_ref, wroot_ref, brel_ref, broot_ref,
               y2_ref, aux_ref):
    a = aggp_ref[0][:N] + aggp_ref[1][:N]
    cnt = a[:, H:H + 1]
    out1 = a[:, :H] / jnp.maximum(cnt, 1.0) + z1_ref[...]
    nrm = jnp.sqrt(jnp.sum(out1 * out1, axis=1, keepdims=True))
    h = jnp.maximum(out1 / jnp.maximum(nrm, 1e-12), 0.0)
    y2 = lax.dot_general(h, wrel_ref[...], (((1,), (1,)), ((), ())),
                         preferred_element_type=jnp.float32)
    y2_ref[...] = jnp.concatenate(
        [y2, jnp.zeros((N, W2AUG - C), jnp.float32)], axis=1)
    z2 = lax.dot_general(h, wroot_ref[...], (((1,), (1,)), ((), ())),
                         preferred_element_type=jnp.float32)
    z2 = z2 + brel_ref[...] + broot_ref[...]
    aux_ref[...] = jnp.concatenate(
        [z2, cnt, jnp.zeros((N, W2AUG - C - 1), jnp.float32)], axis=1)


_tc_b = pl.pallas_call(
    _tc_b_body,
    out_shape=(jax.ShapeDtypeStruct((N, W2AUG), jnp.float32),
               jax.ShapeDtypeStruct((N, W2AUG), jnp.float32)),
)


def _tc_c_body(agg2p_ref, aux_ref, out_ref):
    a = agg2p_ref[0][:N] + agg2p_ref[1][:N]
    aux = aux_ref[...]
    cnt = aux[:, C:C + 1]
    o = a[:, :C] / jnp.maximum(cnt, 1.0) + aux[:, :C]
    nrm = jnp.sqrt(jnp.sum(o * o, axis=1, keepdims=True))
    o = o / jnp.maximum(nrm, 1e-12)
    m = jnp.max(o, axis=1, keepdims=True)
    ls = (o - m) - jnp.log(jnp.sum(jnp.exp(o - m), axis=1, keepdims=True))
    out_ref[...] = jnp.concatenate(
        [ls, jnp.zeros((N, W2AUG - C), jnp.float32)], axis=1)


_tc_c = pl.pallas_call(
    _tc_c_body,
    out_shape=jax.ShapeDtypeStruct((N, W2AUG), jnp.float32),
)


def _sc1_body(y1_hbm, src_hbm, dst_hbm, w_hbm, zero_hbm, out_hbm,
              src_v, dst_v, w_v, rows_v, shared):
    c = lax.axis_index("c")
    s = lax.axis_index("s")
    wid = s * NC + c
    pltpu.sync_copy(src_hbm.at[wid], src_v)
    pltpu.sync_copy(dst_hbm.at[wid], dst_v)
    pltpu.sync_copy(w_hbm.at[wid], w_v)
    pltpu.sync_copy(zero_hbm.at[pl.ds(s * ROWS_PT, ROWS_PT)],
                    shared.at[pl.ds(s * ROWS_PT, ROWS_PT)])
    plsc.subcore_barrier()

    gdn = lax.GatherDimensionNumbers(
        offset_dims=(), collapsed_slice_dims=(0,), start_index_map=(0,))

    def chunk_body(j, carry):
        pltpu.sync_copy(y1_hbm.at[src_v.at[j]], rows_v)
        for g in range(CH // 16):
            wv = w_v[pl.ds(j * CH + g * 16, 16)]
            for l in range(16):
                e = g * 16 + l
                wb = lax.gather(wv, jnp.full((16, 1), l, jnp.int32),
                                dimension_numbers=gdn, slice_sizes=(1,),
                                mode=lax.GatherScatterMode.PROMISE_IN_BOUNDS)
                for cb in range(H // 16):
                    sl = (e, pl.ds(cb * 16, 16))
                    rows_v[sl] = rows_v[sl] * wb
        pltpu.sync_copy(rows_v, shared.at[dst_v.at[j]], add=True)
        return carry

    lax.fori_loop(0, NCHUNK, chunk_body, 0)
    plsc.subcore_barrier()
    pltpu.sync_copy(shared.at[pl.ds(s * ROWS_PT, ROWS_PT)],
                    out_hbm.at[c, pl.ds(s * ROWS_PT, ROWS_PT)])


import functools


@functools.lru_cache(maxsize=None)
def _sc_calls():
    mesh = plsc.VectorSubcoreMesh(core_axis_name="c", subcore_axis_name="s",
                                  num_cores=NC, num_subcores=NS)
    cparams = pltpu.CompilerParams(use_tc_tiling_on_sc=False)
    sc1 = pl.kernel(
        _sc1_body,
        out_type=jax.ShapeDtypeStruct((NC, NP, W1AUG), jnp.float32),
        mesh=mesh,
        compiler_params=cparams,
        scratch_types=[
            pltpu.VMEM((NCHUNK, CH), jnp.int32),
            pltpu.VMEM((NCHUNK, CH), jnp.int32),
            pltpu.VMEM((EPW,), jnp.float32),
            pltpu.VMEM((CH, W1AUG), jnp.float32),
            pltpu.VMEM_SHARED((NP, W1AUG), jnp.float32),
        ],
    )
    sc2 = pl.kernel(
        _sc2_body,
        out_type=jax.ShapeDtypeStruct((NC, NP, W2AUG), jnp.float32),
        mesh=mesh,
        compiler_params=cparams,
        scratch_types=[
            pltpu.VMEM((NCHUNK, CH), jnp.int32),
            pltpu.VMEM((NCHUNK, CH), jnp.int32),
            pltpu.VMEM((CH, W2AUG), jnp.float32),
            pltpu.VMEM_SHARED((NP, W2AUG), jnp.float32),
        ],
    )
    return sc1, sc2


def _sc2_body(y2_hbm, src_hbm, dst_hbm, zero_hbm, out_hbm,
              src_v, dst_v, rows_v, shared):
    c = lax.axis_index("c")
    s = lax.axis_index("s")
    wid = s * NC + c
    pltpu.sync_copy(src_hbm.at[wid], src_v)
    pltpu.sync_copy(dst_hbm.at[wid], dst_v)
    pltpu.sync_copy(zero_hbm.at[pl.ds(s * ROWS_PT, ROWS_PT)],
                    shared.at[pl.ds(s * ROWS_PT, ROWS_PT)])
    plsc.subcore_barrier()

    def chunk_body(j, carry):
        pltpu.sync_copy(y2_hbm.at[src_v.at[j]], rows_v)
        pltpu.sync_copy(rows_v, shared.at[dst_v.at[j]], add=True)
        return carry

    lax.fori_loop(0, NCHUNK, chunk_body, 0)
    plsc.subcore_barrier()
    pltpu.sync_copy(shared.at[pl.ds(s * ROWS_PT, ROWS_PT)],
                    out_hbm.at[c, pl.ds(s * ROWS_PT, ROWS_PT)])


def kernel(x, edge_index, weight, W1_rel, b1_rel, W1_root, b1_root,
           W2_rel, b2_rel, W2_root, b2_root):
    _sc1, _sc2 = _sc_calls()
    src = edge_index[0].reshape(NW, NCHUNK, CH)
    dst = edge_index[1].reshape(NW, NCHUNK, CH)
    w2 = weight.reshape(NW, EPW)
    y1aug, z1 = _tc_a(x, W1_rel, W1_root,
                      b1_rel.reshape(1, H), b1_root.reshape(1, H))
    agg1 = _sc1(y1aug, src, dst, w2, jnp.zeros((NP, W1AUG), jnp.float32))
    y2aug, aux = _tc_b(agg1, z1, W2_rel, W2_root,
                       b2_rel.reshape(1, C), b2_root.reshape(1, C))
    agg2 = _sc2(y2aug, src, dst, jnp.zeros((NP, W2AUG), jnp.float32))
    out = _tc_c(agg2, aux)
    return out[:, :C]


# speedup vs baseline: 16.5775x; 1.6583x over previous
import jax
import jax.numpy as jnp
from jax import lax
from jax.experimental import pallas as pl
from jax.experimental.pallas import tpu as pltpu
from jax.experimental.pallas import tpu_sc as plsc

N, E, D, H, C = 10000, 320000, 128, 32, 5
W1AUG = 48
W2AUG = 16
NC, NS = 2, 16
NW = NC * NS
EPW = E // NW
CH = 400
NCHUNK = EPW // CH
CH2 = 2000
NCHUNK2 = EPW // CH2
NP = 10240
ROWS_PT = NP // NS


def _tc_a_body(x_ref, wrel_ref, wroot_ref, brel_ref, broot_ref, y1_ref, z1_ref):
    x = x_ref[...]
    y = lax.dot_general(x, wrel_ref[...], (((1,), (1,)), ((), ())),
                        preferred_element_type=jnp.float32)
    ones = jnp.ones((N, 1), jnp.float32)
    pad = jnp.zeros((N, W1AUG - H - 1), jnp.float32)
    y1_ref[...] = jnp.concatenate([y, ones, pad], axis=1)
    z = lax.dot_general(x, wroot_ref[...], (((1,), (1,)), ((), ())),
                        preferred_element_type=jnp.float32)
    z1_ref[...] = z + brel_ref[...] + broot_ref[...]


_tc_a = pl.pallas_call(
    _tc_a_body,
    out_shape=(jax.ShapeDtypeStruct((N, W1AUG), jnp.float32),
               jax.ShapeDtypeStruct((N, H), jnp.float32)),
)


def _tc_b_body(aggp_ref, z1_ref, wrel_ref, wroot_ref, brel_ref, broot_ref,
               y2_ref, aux_ref):
    a = aggp_ref[0][:N] + aggp_ref[1][:N]
    cnt = a[:, H:H + 1]
    out1 = a[:, :H] / jnp.maximum(cnt, 1.0) + z1_ref[...]
    nrm = jnp.sqrt(jnp.sum(out1 * out1, axis=1, keepdims=True))
    h = jnp.maximum(out1 / jnp.maximum(nrm, 1e-12), 0.0)
    y2 = lax.dot_general(h, wrel_ref[...], (((1,), (1,)), ((), ())),
                         preferred_element_type=jnp.float32)
    y2_ref[...] = jnp.concatenate(
        [y2, jnp.zeros((N, W2AUG - C), jnp.float32)], axis=1)
    z2 = lax.dot_general(h, wroot_ref[...], (((1,), (1,)), ((), ())),
                         preferred_element_type=jnp.float32)
    z2 = z2 + brel_ref[...] + broot_ref[...]
    aux_ref[...] = jnp.concatenate(
        [z2, cnt, jnp.zeros((N, W2AUG - C - 1), jnp.float32)], axis=1)


_tc_b = pl.pallas_call(
    _tc_b_body,
    out_shape=(jax.ShapeDtypeStruct((N, W2AUG), jnp.float32),
               jax.ShapeDtypeStruct((N, W2AUG), jnp.float32)),
)


def _tc_c_body(agg2p_ref, aux_ref, out_ref):
    a = agg2p_ref[0][:N] + agg2p_ref[1][:N]
    aux = aux_ref[...]
    cnt = aux[:, C:C + 1]
    o = a[:, :C] / jnp.maximum(cnt, 1.0) + aux[:, :C]
    nrm = jnp.sqrt(jnp.sum(o * o, axis=1, keepdims=True))
    o = o / jnp.maximum(nrm, 1e-12)
    m = jnp.max(o, axis=1, keepdims=True)
    ls = (o - m) - jnp.log(jnp.sum(jnp.exp(o - m), axis=1, keepdims=True))
    out_ref[...] = jnp.concatenate(
        [ls, jnp.zeros((N, W2AUG - C), jnp.float32)], axis=1)


_tc_c = pl.pallas_call(
    _tc_c_body,
    out_shape=jax.ShapeDtypeStruct((N, W2AUG), jnp.float32),
)


def _sc1_body(y1_hbm, src_hbm, dst_hbm, w_hbm, zero_hbm, out_hbm,
              src_v, dst_v, w_v, rows_v, shared):
    c = lax.axis_index("c")
    s = lax.axis_index("s")
    wid = s * NC + c
    pltpu.sync_copy(src_hbm.at[wid], src_v)
    pltpu.sync_copy(dst_hbm.at[wid], dst_v)
    pltpu.sync_copy(w_hbm.at[wid], w_v)
    pltpu.sync_copy(zero_hbm.at[pl.ds(s * ROWS_PT, ROWS_PT)],
                    shared.at[pl.ds(s * ROWS_PT, ROWS_PT)])
    plsc.subcore_barrier()

    gdn = lax.GatherDimensionNumbers(
        offset_dims=(), collapsed_slice_dims=(0,), start_index_map=(0,))

    def chunk_body(j, carry):
        pltpu.sync_copy(y1_hbm.at[src_v.at[j]], rows_v)

        def grp_body(g, carry2):
            wv = w_v[pl.ds(j * CH + g * 16, 16)]
            for l in range(16):
                wb = lax.gather(wv, jnp.full((16, 1), l, jnp.int32),
                                dimension_numbers=gdn, slice_sizes=(1,),
                                mode=lax.GatherScatterMode.PROMISE_IN_BOUNDS)
                e = g * 16 + l
                for cb in range(H // 16):
                    sl = (e, pl.ds(cb * 16, 16))
                    rows_v[sl] = rows_v[sl] * wb
            return carry2

        lax.fori_loop(0, CH // 16, grp_body, 0)
        pltpu.sync_copy(rows_v, shared.at[dst_v.at[j]], add=True)
        return carry

    lax.fori_loop(0, NCHUNK, chunk_body, 0)
    plsc.subcore_barrier()
    pltpu.sync_copy(shared.at[pl.ds(s * ROWS_PT, ROWS_PT)],
                    out_hbm.at[c, pl.ds(s * ROWS_PT, ROWS_PT)])


import functools


@functools.lru_cache(maxsize=None)
def _sc_calls():
    mesh = plsc.VectorSubcoreMesh(core_axis_name="c", subcore_axis_name="s",
                                  num_cores=NC, num_subcores=NS)
    cparams = pltpu.CompilerParams(use_tc_tiling_on_sc=False)
    sc1 = pl.kernel(
        _sc1_body,
        out_type=jax.ShapeDtypeStruct((NC, NP, W1AUG), jnp.float32),
        mesh=mesh,
        compiler_params=cparams,
        scratch_types=[
            pltpu.VMEM((NCHUNK, CH), jnp.int32),
            pltpu.VMEM((NCHUNK, CH), jnp.int32),
            pltpu.VMEM((EPW,), jnp.float32),
            pltpu.VMEM((CH, W1AUG), jnp.float32),
            pltpu.VMEM_SHARED((NP, W1AUG), jnp.float32),
        ],
    )
    sc2 = pl.kernel(
        _sc2_body,
        out_type=jax.ShapeDtypeStruct((NC, NP, W2AUG), jnp.float32),
        mesh=mesh,
        compiler_params=cparams,
        scratch_types=[
            pltpu.VMEM((NCHUNK2, CH2), jnp.int32),
            pltpu.VMEM((NCHUNK2, CH2), jnp.int32),
            pltpu.VMEM((CH2, W2AUG), jnp.float32),
            pltpu.VMEM_SHARED((NP, W2AUG), jnp.float32),
        ],
    )
    return sc1, sc2


def _sc2_body(y2_hbm, src_hbm, dst_hbm, zero_hbm, out_hbm,
              src_v, dst_v, rows_v, shared):
    c = lax.axis_index("c")
    s = lax.axis_index("s")
    wid = s * NC + c
    pltpu.sync_copy(src_hbm.at[wid], src_v)
    pltpu.sync_copy(dst_hbm.at[wid], dst_v)
    pltpu.sync_copy(zero_hbm.at[pl.ds(s * ROWS_PT, ROWS_PT)],
                    shared.at[pl.ds(s * ROWS_PT, ROWS_PT)])
    plsc.subcore_barrier()

    def chunk_body(j, carry):
        pltpu.sync_copy(y2_hbm.at[src_v.at[j]], rows_v)
        pltpu.sync_copy(rows_v, shared.at[dst_v.at[j]], add=True)
        return carry

    lax.fori_loop(0, NCHUNK2, chunk_body, 0)
    plsc.subcore_barrier()
    pltpu.sync_copy(shared.at[pl.ds(s * ROWS_PT, ROWS_PT)],
                    out_hbm.at[c, pl.ds(s * ROWS_PT, ROWS_PT)])


def kernel(x, edge_index, weight, W1_rel, b1_rel, W1_root, b1_root,
           W2_rel, b2_rel, W2_root, b2_root):
    _sc1, _sc2 = _sc_calls()
    src = edge_index[0].reshape(NW, NCHUNK, CH)
    dst = edge_index[1].reshape(NW, NCHUNK, CH)
    src2 = edge_index[0].reshape(NW, NCHUNK2, CH2)
    dst2 = edge_index[1].reshape(NW, NCHUNK2, CH2)
    w2 = weight.reshape(NW, EPW)
    y1aug, z1 = _tc_a(x, W1_rel, W1_root,
                      b1_rel.reshape(1, H), b1_root.reshape(1, H))
    agg1 = _sc1(y1aug, src, dst, w2, jnp.zeros((NP, W1AUG), jnp.float32))
    y2aug, aux = _tc_b(agg1, z1, W2_rel, W2_root,
                       b2_rel.reshape(1, C), b2_root.reshape(1, C))
    agg2 = _sc2(y2aug, src2, dst2, jnp.zeros((NP, W2AUG), jnp.float32))
    out = _tc_c(agg2, aux)
    return out[:, :C]


# speedup vs baseline: 16.7473x; 1.0102x over previous
import jax
import jax.numpy as jnp
from jax import lax
from jax.experimental import pallas as pl
from jax.experimental.pallas import tpu as pltpu
from jax.experimental.pallas import tpu_sc as plsc

N, E, D, H, C = 10000, 320000, 128, 32, 5
W1AUG = 40
W2AUG = 16
NC, NS = 2, 16
NW = NC * NS
EPW = E // NW
CH = 400
NCHUNK = EPW // CH
CH2 = 2000
NCHUNK2 = EPW // CH2
NP = 10240
ROWS_PT = NP // NS


def _tc_a_body(x_ref, wrel_ref, wroot_ref, brel_ref, broot_ref, y1_ref, z1_ref):
    x = x_ref[...]
    y = lax.dot_general(x, wrel_ref[...], (((1,), (1,)), ((), ())),
                        preferred_element_type=jnp.float32)
    ones = jnp.ones((N, 1), jnp.float32)
    pad = jnp.zeros((N, W1AUG - H - 1), jnp.float32)
    y1_ref[...] = jnp.concatenate([y, ones, pad], axis=1)
    z = lax.dot_general(x, wroot_ref[...], (((1,), (1,)), ((), ())),
                        preferred_element_type=jnp.float32)
    z1_ref[...] = z + brel_ref[...] + broot_ref[...]


_tc_a = pl.pallas_call(
    _tc_a_body,
    out_shape=(jax.ShapeDtypeStruct((N, W1AUG), jnp.float32),
               jax.ShapeDtypeStruct((N, H), jnp.float32)),
)


def _tc_b_body(aggp_ref, z1_ref, wrel_ref, wroot_ref, brel_ref, broot_ref,
               y2_ref, aux_ref):
    a = aggp_ref[0][:N] + aggp_ref[1][:N]
    cnt = a[:, H:H + 1]
    out1 = a[:, :H] / jnp.maximum(cnt, 1.0) + z1_ref[...]
    nrm = jnp.sqrt(jnp.sum(out1 * out1, axis=1, keepdims=True))
    h = jnp.maximum(out1 / jnp.maximum(nrm, 1e-12), 0.0)
    y2 = lax.dot_general(h, wrel_ref[...], (((1,), (1,)), ((), ())),
                         preferred_element_type=jnp.float32)
    y2_ref[...] = jnp.concatenate(
        [y2, jnp.zeros((N, W2AUG - C), jnp.float32)], axis=1)
    z2 = lax.dot_general(h, wroot_ref[...], (((1,), (1,)), ((), ())),
                         preferred_element_type=jnp.float32)
    z2 = z2 + brel_ref[...] + broot_ref[...]
    aux_ref[...] = jnp.concatenate(
        [z2, cnt, jnp.zeros((N, W2AUG - C - 1), jnp.float32)], axis=1)


_tc_b = pl.pallas_call(
    _tc_b_body,
    out_shape=(jax.ShapeDtypeStruct((N, W2AUG), jnp.float32),
               jax.ShapeDtypeStruct((N, W2AUG), jnp.float32)),
)


def _tc_c_body(agg2p_ref, aux_ref, out_ref):
    a = agg2p_ref[0][:N] + agg2p_ref[1][:N]
    aux = aux_ref[...]
    cnt = aux[:, C:C + 1]
    o = a[:, :C] / jnp.maximum(cnt, 1.0) + aux[:, :C]
    nrm = jnp.sqrt(jnp.sum(o * o, axis=1, keepdims=True))
    o = o / jnp.maximum(nrm, 1e-12)
    m = jnp.max(o, axis=1, keepdims=True)
    ls = (o - m) - jnp.log(jnp.sum(jnp.exp(o - m), axis=1, keepdims=True))
    out_ref[...] = jnp.concatenate(
        [ls, jnp.zeros((N, W2AUG - C), jnp.float32)], axis=1)


_tc_c = pl.pallas_call(
    _tc_c_body,
    out_shape=jax.ShapeDtypeStruct((N, W2AUG), jnp.float32),
)


def _sc1_body(y1_hbm, src_hbm, dst_hbm, w_hbm, out_hbm,
              src_v, dst_v, w_v, rows_v, shared):
    c = lax.axis_index("c")
    s = lax.axis_index("s")
    wid = s * NC + c
    rows_v[...] = jnp.zeros((CH, W1AUG), jnp.float32)
    pltpu.sync_copy(rows_v, shared.at[pl.ds(s * ROWS_PT, CH)])
    pltpu.sync_copy(rows_v.at[pl.ds(0, ROWS_PT - CH)],
                    shared.at[pl.ds(s * ROWS_PT + CH, ROWS_PT - CH)])
    pltpu.sync_copy(src_hbm.at[wid], src_v)
    pltpu.sync_copy(dst_hbm.at[wid], dst_v)
    pltpu.sync_copy(w_hbm.at[wid], w_v)
    plsc.subcore_barrier()

    gdn = lax.GatherDimensionNumbers(
        offset_dims=(), collapsed_slice_dims=(0,), start_index_map=(0,))

    def chunk_body(j, carry):
        pltpu.sync_copy(y1_hbm.at[src_v.at[j]], rows_v)

        def grp_body(g, carry2):
            wv = w_v[pl.ds(j * CH + g * 16, 16)]
            for l in range(16):
                wb = lax.gather(wv, jnp.full((16, 1), l, jnp.int32),
                                dimension_numbers=gdn, slice_sizes=(1,),
                                mode=lax.GatherScatterMode.PROMISE_IN_BOUNDS)
                e = g * 16 + l
                for cb in range(H // 16):
                    sl = (e, pl.ds(cb * 16, 16))
                    rows_v[sl] = rows_v[sl] * wb
            return carry2

        lax.fori_loop(0, CH // 16, grp_body, 0)
        pltpu.sync_copy(rows_v, shared.at[dst_v.at[j]], add=True)
        return carry

    lax.fori_loop(0, NCHUNK, chunk_body, 0)
    plsc.subcore_barrier()
    pltpu.sync_copy(shared.at[pl.ds(s * ROWS_PT, ROWS_PT)],
                    out_hbm.at[c, pl.ds(s * ROWS_PT, ROWS_PT)])


import functools


@functools.lru_cache(maxsize=None)
def _sc_calls():
    mesh = plsc.VectorSubcoreMesh(core_axis_name="c", subcore_axis_name="s",
                                  num_cores=NC, num_subcores=NS)
    cparams = pltpu.CompilerParams(use_tc_tiling_on_sc=False)
    sc1 = pl.kernel(
        _sc1_body,
        out_type=jax.ShapeDtypeStruct((NC, NP, W1AUG), jnp.float32),
        mesh=mesh,
        compiler_params=cparams,
        scratch_types=[
            pltpu.VMEM((NCHUNK, CH), jnp.int32),
            pltpu.VMEM((NCHUNK, CH), jnp.int32),
            pltpu.VMEM((EPW,), jnp.float32),
            pltpu.VMEM((CH, W1AUG), jnp.float32),
            pltpu.VMEM_SHARED((NP, W1AUG), jnp.float32),
        ],
    )
    sc2 = pl.kernel(
        _sc2_body,
        out_type=jax.ShapeDtypeStruct((NC, NP, W2AUG), jnp.float32),
        mesh=mesh,
        compiler_params=cparams,
        scratch_types=[
            pltpu.VMEM((NCHUNK2, CH2), jnp.int32),
            pltpu.VMEM((NCHUNK2, CH2), jnp.int32),
            pltpu.VMEM((CH2, W2AUG), jnp.float32),
            pltpu.VMEM_SHARED((NP, W2AUG), jnp.float32),
        ],
    )
    return sc1, sc2


def _sc2_body(y2_hbm, src_hbm, dst_hbm, out_hbm,
              src_v, dst_v, rows_v, shared):
    c = lax.axis_index("c")
    s = lax.axis_index("s")
    wid = s * NC + c
    rows_v[...] = jnp.zeros((CH2, W2AUG), jnp.float32)
    pltpu.sync_copy(rows_v.at[pl.ds(0, ROWS_PT)],
                    shared.at[pl.ds(s * ROWS_PT, ROWS_PT)])
    pltpu.sync_copy(src_hbm.at[wid], src_v)
    pltpu.sync_copy(dst_hbm.at[wid], dst_v)
    plsc.subcore_barrier()

    def chunk_body(j, carry):
        pltpu.sync_copy(y2_hbm.at[src_v.at[j]], rows_v)
        pltpu.sync_copy(rows_v, shared.at[dst_v.at[j]], add=True)
        return carry

    lax.fori_loop(0, NCHUNK2, chunk_body, 0)
    plsc.subcore_barrier()
    pltpu.sync_copy(shared.at[pl.ds(s * ROWS_PT, ROWS_PT)],
                    out_hbm.at[c, pl.ds(s * ROWS_PT, ROWS_PT)])


def kernel(x, edge_index, weight, W1_rel, b1_rel, W1_root, b1_root,
           W2_rel, b2_rel, W2_root, b2_root):
    _sc1, _sc2 = _sc_calls()
    src = edge_index[0].reshape(NW, NCHUNK, CH)
    dst = edge_index[1].reshape(NW, NCHUNK, CH)
    src2 = edge_index[0].reshape(NW, NCHUNK2, CH2)
    dst2 = edge_index[1].reshape(NW, NCHUNK2, CH2)
    w2 = weight.reshape(NW, EPW)
    y1aug, z1 = _tc_a(x, W1_rel, W1_root,
                      b1_rel.reshape(1, H), b1_root.reshape(1, H))
    agg1 = _sc1(y1aug, src, dst, w2)
    y2aug, aux = _tc_b(agg1, z1, W2_rel, W2_root,
                       b2_rel.reshape(1, C), b2_root.reshape(1, C))
    agg2 = _sc2(y2aug, src2, dst2)
    out = _tc_c(agg2, aux)
    return out[:, :C]


# speedup vs baseline: 20.0478x; 1.1971x over previous
import jax
import jax.numpy as jnp
from jax import lax
from jax.experimental import pallas as pl
from jax.experimental.pallas import tpu as pltpu
from jax.experimental.pallas import tpu_sc as plsc

N, E, D, H, C = 10000, 320000, 128, 32, 5
W1AUG = 40
W2AUG = 8
AUXW = 16
NC, NS = 2, 16
NW = NC * NS
EPW = E // NW
CH = 400
NCHUNK = EPW // CH
CH2 = 2000
NCHUNK2 = EPW // CH2
NP = 10240
ROWS_PT = NP // NS


def _tc_a_body(x_ref, wrel_ref, wroot_ref, brel_ref, broot_ref, y1_ref, z1_ref):
    x = x_ref[...]
    y = lax.dot_general(x, wrel_ref[...], (((1,), (1,)), ((), ())),
                        preferred_element_type=jnp.float32)
    ones = jnp.ones((N, 1), jnp.float32)
    pad = jnp.zeros((N, W1AUG - H - 1), jnp.float32)
    y1_ref[...] = jnp.concatenate([y, ones, pad], axis=1)
    z = lax.dot_general(x, wroot_ref[...], (((1,), (1,)), ((), ())),
                        preferred_element_type=jnp.float32)
    z1_ref[...] = z + brel_ref[...] + broot_ref[...]


_tc_a = pl.pallas_call(
    _tc_a_body,
    out_shape=(jax.ShapeDtypeStruct((N, W1AUG), jnp.float32),
               jax.ShapeDtypeStruct((N, H), jnp.float32)),
)


def _tc_b_body(aggp_ref, z1_ref, wrel_ref, wroot_ref, brel_ref, broot_ref,
               y2_ref, aux_ref):
    a = aggp_ref[0][:N] + aggp_ref[1][:N]
    cnt = a[:, H:H + 1]
    out1 = a[:, :H] / jnp.maximum(cnt, 1.0) + z1_ref[...]
    nrm = jnp.sqrt(jnp.sum(out1 * out1, axis=1, keepdims=True))
    h = jnp.maximum(out1 / jnp.maximum(nrm, 1e-12), 0.0)
    y2 = lax.dot_general(h, wrel_ref[...], (((1,), (1,)), ((), ())),
                         preferred_element_type=jnp.float32)
    y2_ref[...] = jnp.concatenate(
        [y2, jnp.zeros((N, W2AUG - C), jnp.float32)], axis=1)
    z2 = lax.dot_general(h, wroot_ref[...], (((1,), (1,)), ((), ())),
                         preferred_element_type=jnp.float32)
    z2 = z2 + brel_ref[...] + broot_ref[...]
    aux_ref[...] = jnp.concatenate(
        [z2, cnt, jnp.zeros((N, AUXW - C - 1), jnp.float32)], axis=1)


_tc_b = pl.pallas_call(
    _tc_b_body,
    out_shape=(jax.ShapeDtypeStruct((N, W2AUG), jnp.float32),
               jax.ShapeDtypeStruct((N, AUXW), jnp.float32)),
)


def _tc_c_body(agg2p_ref, aux_ref, out_ref):
    a = agg2p_ref[0][:N] + agg2p_ref[1][:N]
    aux = aux_ref[...]
    cnt = aux[:, C:C + 1]
    o = a[:, :C] / jnp.maximum(cnt, 1.0) + aux[:, :C]
    nrm = jnp.sqrt(jnp.sum(o * o, axis=1, keepdims=True))
    o = o / jnp.maximum(nrm, 1e-12)
    m = jnp.max(o, axis=1, keepdims=True)
    ls = (o - m) - jnp.log(jnp.sum(jnp.exp(o - m), axis=1, keepdims=True))
    out_ref[...] = ls


_tc_c = pl.pallas_call(
    _tc_c_body,
    out_shape=jax.ShapeDtypeStruct((N, C), jnp.float32),
)


def _sc1_body(y1_hbm, src_hbm, dst_hbm, w_hbm, out_hbm,
              src_v, dst_v, w_v, rows_a, rows_b, shared, sem_a, sem_b):
    c = lax.axis_index("c")
    s = lax.axis_index("s")
    wid = s * NC + c
    rows_a[...] = jnp.zeros((CH, W1AUG), jnp.float32)
    pltpu.sync_copy(rows_a, shared.at[pl.ds(s * ROWS_PT, CH)])
    pltpu.sync_copy(rows_a.at[pl.ds(0, ROWS_PT - CH)],
                    shared.at[pl.ds(s * ROWS_PT + CH, ROWS_PT - CH)])
    pltpu.sync_copy(src_hbm.at[wid], src_v)
    pltpu.sync_copy(dst_hbm.at[wid], dst_v)
    pltpu.sync_copy(w_hbm.at[wid], w_v)
    plsc.subcore_barrier()

    gdn = lax.GatherDimensionNumbers(
        offset_dims=(), collapsed_slice_dims=(0,), start_index_map=(0,))
    bufs = (rows_a, rows_b)
    sems = (sem_a, sem_b)

    def scale(rows_v, j):
        def grp_body(g, carry2):
            wv = w_v[pl.ds(j * CH + g * 16, 16)]
            for l in range(16):
                wb = lax.gather(wv, jnp.full((16, 1), l, jnp.int32),
                                dimension_numbers=gdn, slice_sizes=(1,),
                                mode=lax.GatherScatterMode.PROMISE_IN_BOUNDS)
                e = g * 16 + l
                for cb in range(H // 16):
                    sl = (e, pl.ds(cb * 16, 16))
                    rows_v[sl] = rows_v[sl] * wb
            return carry2

        lax.fori_loop(0, CH // 16, grp_body, 0)

    pltpu.async_copy(y1_hbm.at[src_v.at[0]], rows_a, sem_a)
    for j in range(NCHUNK):
        b = j % 2
        if j + 1 < NCHUNK:
            pltpu.async_copy(y1_hbm.at[src_v.at[j + 1]], bufs[1 - b],
                             sems[1 - b])
        pltpu.make_async_copy(y1_hbm.at[pl.ds(0, CH)], bufs[b],
                              sems[b]).wait()
        scale(bufs[b], j)
        pltpu.sync_copy(bufs[b], shared.at[dst_v.at[j]], add=True)
    plsc.subcore_barrier()
    pltpu.sync_copy(shared.at[pl.ds(s * ROWS_PT, ROWS_PT)],
                    out_hbm.at[c, pl.ds(s * ROWS_PT, ROWS_PT)])


import functools


@functools.lru_cache(maxsize=None)
def _sc_calls():
    mesh = plsc.VectorSubcoreMesh(core_axis_name="c", subcore_axis_name="s",
                                  num_cores=NC, num_subcores=NS)
    cparams = pltpu.CompilerParams(use_tc_tiling_on_sc=False)
    sc1 = pl.kernel(
        _sc1_body,
        out_type=jax.ShapeDtypeStruct((NC, NP, W1AUG), jnp.float32),
        mesh=mesh,
        compiler_params=cparams,
        scratch_types=[
            pltpu.VMEM((NCHUNK, CH), jnp.int32),
            pltpu.VMEM((NCHUNK, CH), jnp.int32),
            pltpu.VMEM((EPW,), jnp.float32),
            pltpu.VMEM((CH, W1AUG), jnp.float32),
            pltpu.VMEM((CH, W1AUG), jnp.float32),
            pltpu.VMEM_SHARED((NP, W1AUG), jnp.float32),
            pltpu.SemaphoreType.DMA,
            pltpu.SemaphoreType.DMA,
        ],
    )
    sc2 = pl.kernel(
        _sc2_body,
        out_type=jax.ShapeDtypeStruct((NC, NP, W2AUG), jnp.float32),
        mesh=mesh,
        compiler_params=cparams,
        scratch_types=[
            pltpu.VMEM((NCHUNK2, CH2), jnp.int32),
            pltpu.VMEM((NCHUNK2, CH2), jnp.int32),
            pltpu.VMEM((CH2, W2AUG), jnp.float32),
            pltpu.VMEM((CH2, W2AUG), jnp.float32),
            pltpu.VMEM_SHARED((NP, W2AUG), jnp.float32),
            pltpu.SemaphoreType.DMA,
            pltpu.SemaphoreType.DMA,
        ],
    )
    return sc1, sc2


def _sc2_body(y2_hbm, src_hbm, dst_hbm, out_hbm,
              src_v, dst_v, rows_a, rows_b, shared, sem_a, sem_b):
    c = lax.axis_index("c")
    s = lax.axis_index("s")
    wid = s * NC + c
    rows_a[pl.ds(0, ROWS_PT), :] = jnp.zeros((ROWS_PT, W2AUG), jnp.float32)
    pltpu.sync_copy(rows_a.at[pl.ds(0, ROWS_PT)],
                    shared.at[pl.ds(s * ROWS_PT, ROWS_PT)])
    pltpu.sync_copy(src_hbm.at[wid], src_v)
    pltpu.sync_copy(dst_hbm.at[wid], dst_v)
    plsc.subcore_barrier()

    bufs = (rows_a, rows_b)
    sems = (sem_a, sem_b)
    pltpu.async_copy(y2_hbm.at[src_v.at[0]], rows_a, sem_a)
    for j in range(NCHUNK2):
        b = j % 2
        if j + 1 < NCHUNK2:
            pltpu.async_copy(y2_hbm.at[src_v.at[j + 1]], bufs[1 - b],
                             sems[1 - b])
        pltpu.make_async_copy(y2_hbm.at[pl.ds(0, CH2)], bufs[b],
                              sems[b]).wait()
        pltpu.sync_copy(bufs[b], shared.at[dst_v.at[j]], add=True)
    plsc.subcore_barrier()
    pltpu.sync_copy(shared.at[pl.ds(s * ROWS_PT, ROWS_PT)],
                    out_hbm.at[c, pl.ds(s * ROWS_PT, ROWS_PT)])


def kernel(x, edge_index, weight, W1_rel, b1_rel, W1_root, b1_root,
           W2_rel, b2_rel, W2_root, b2_root):
    _sc1, _sc2 = _sc_calls()
    src = edge_index[0].reshape(NW, NCHUNK, CH)
    dst = edge_index[1].reshape(NW, NCHUNK, CH)
    src2 = edge_index[0].reshape(NW, NCHUNK2, CH2)
    dst2 = edge_index[1].reshape(NW, NCHUNK2, CH2)
    w2 = weight.reshape(NW, EPW)
    y1aug, z1 = _tc_a(x, W1_rel, W1_root,
                      b1_rel.reshape(1, H), b1_root.reshape(1, H))
    agg1 = _sc1(y1aug, src, dst, w2)
    y2aug, aux = _tc_b(agg1, z1, W2_rel, W2_root,
                       b2_rel.reshape(1, C), b2_root.reshape(1, C))
    agg2 = _sc2(y2aug, src2, dst2)
    return _tc_c(agg2, aux)


# speedup vs baseline: 20.8243x; 1.0387x over previous
import jax
import jax.numpy as jnp
from jax import lax
from jax.experimental import pallas as pl
from jax.experimental.pallas import tpu as pltpu
from jax.experimental.pallas import tpu_sc as plsc

N, E, D, H, C = 10000, 320000, 128, 32, 5
W1AUG = 40
W2AUG = 8
AUXW = 16
NC, NS = 2, 16
NW = NC * NS
EPW = E // NW
CH = 400
NCHUNK = EPW // CH
CH2 = 2000
NCHUNK2 = EPW // CH2
NP = 10240
ROWS_PT = NP // NS


def _tc_a_body(x_ref, wrel_ref, wroot_ref, brel_ref, broot_ref, y1_ref, z1_ref):
    x = x_ref[...]
    y = lax.dot_general(x, wrel_ref[...], (((1,), (1,)), ((), ())),
                        preferred_element_type=jnp.float32)
    ones = jnp.ones((N, 1), jnp.float32)
    pad = jnp.zeros((N, W1AUG - H - 1), jnp.float32)
    y1_ref[...] = jnp.concatenate([y, ones, pad], axis=1)
    z = lax.dot_general(x, wroot_ref[...], (((1,), (1,)), ((), ())),
                        preferred_element_type=jnp.float32)
    z1_ref[...] = z + brel_ref[...] + broot_ref[...]


_tc_a = pl.pallas_call(
    _tc_a_body,
    out_shape=(jax.ShapeDtypeStruct((N, W1AUG), jnp.float32),
               jax.ShapeDtypeStruct((N, H), jnp.float32)),
)


def _tc_b_body(aggp_ref, z1_ref, wrel_ref, wroot_ref, brel_ref, broot_ref,
               y2_ref, aux_ref):
    a = aggp_ref[0][:N] + aggp_ref[1][:N]
    cnt = a[:, H:H + 1]
    out1 = a[:, :H] / jnp.maximum(cnt, 1.0) + z1_ref[...]
    nrm = jnp.sqrt(jnp.sum(out1 * out1, axis=1, keepdims=True))
    h = jnp.maximum(out1 / jnp.maximum(nrm, 1e-12), 0.0)
    y2 = lax.dot_general(h, wrel_ref[...], (((1,), (1,)), ((), ())),
                         preferred_element_type=jnp.float32)
    y2_ref[...] = jnp.concatenate(
        [y2, jnp.zeros((N, W2AUG - C), jnp.float32)], axis=1)
    z2 = lax.dot_general(h, wroot_ref[...], (((1,), (1,)), ((), ())),
                         preferred_element_type=jnp.float32)
    z2 = z2 + brel_ref[...] + broot_ref[...]
    aux_ref[...] = jnp.concatenate(
        [z2, cnt, jnp.zeros((N, AUXW - C - 1), jnp.float32)], axis=1)


_tc_b = pl.pallas_call(
    _tc_b_body,
    out_shape=(jax.ShapeDtypeStruct((N, W2AUG), jnp.float32),
               jax.ShapeDtypeStruct((N, AUXW), jnp.float32)),
)


def _tc_c_body(agg2p_ref, aux_ref, out_ref):
    a = agg2p_ref[0][:N] + agg2p_ref[1][:N]
    aux = aux_ref[...]
    cnt = aux[:, C:C + 1]
    o = a[:, :C] / jnp.maximum(cnt, 1.0) + aux[:, :C]
    nrm = jnp.sqrt(jnp.sum(o * o, axis=1, keepdims=True))
    o = o / jnp.maximum(nrm, 1e-12)
    m = jnp.max(o, axis=1, keepdims=True)
    ls = (o - m) - jnp.log(jnp.sum(jnp.exp(o - m), axis=1, keepdims=True))
    out_ref[...] = ls


_tc_c = pl.pallas_call(
    _tc_c_body,
    out_shape=jax.ShapeDtypeStruct((N, C), jnp.float32),
)


def _sc1_body(y1_hbm, src_hbm, dst_hbm, w_hbm, out_hbm,
              src_v, dst_v, w_v, rows_a, rows_b, rows_c, shared,
              gsem_a, gsem_b, gsem_c, ssem_a, ssem_b, ssem_c):
    c = lax.axis_index("c")
    s = lax.axis_index("s")
    wid = s * NC + c
    rows_a[...] = jnp.zeros((CH, W1AUG), jnp.float32)
    pltpu.sync_copy(rows_a, shared.at[pl.ds(s * ROWS_PT, CH)])
    pltpu.sync_copy(rows_a.at[pl.ds(0, ROWS_PT - CH)],
                    shared.at[pl.ds(s * ROWS_PT + CH, ROWS_PT - CH)])
    pltpu.sync_copy(src_hbm.at[wid], src_v)
    pltpu.sync_copy(dst_hbm.at[wid], dst_v)
    pltpu.sync_copy(w_hbm.at[wid], w_v)
    plsc.subcore_barrier()

    gdn = lax.GatherDimensionNumbers(
        offset_dims=(), collapsed_slice_dims=(0,), start_index_map=(0,))
    bufs = (rows_a, rows_b, rows_c)
    gsems = (gsem_a, gsem_b, gsem_c)
    ssems = (ssem_a, ssem_b, ssem_c)

    def scale(rows_v, j):
        def grp_body(g, carry2):
            wv = w_v[pl.ds(j * CH + g * 16, 16)]
            for l in range(16):
                wb = lax.gather(wv, jnp.full((16, 1), l, jnp.int32),
                                dimension_numbers=gdn, slice_sizes=(1,),
                                mode=lax.GatherScatterMode.PROMISE_IN_BOUNDS)
                e = g * 16 + l
                for cb in range(H // 16):
                    sl = (e, pl.ds(cb * 16, 16))
                    rows_v[sl] = rows_v[sl] * wb
            return carry2

        lax.fori_loop(0, CH // 16, grp_body, 0)

    def wait_dma(buf, sem):
        pltpu.make_async_copy(y1_hbm.at[pl.ds(0, CH)], buf, sem).wait()

    pltpu.async_copy(y1_hbm.at[src_v.at[0]], bufs[0], gsems[0])
    pltpu.async_copy(y1_hbm.at[src_v.at[1]], bufs[1], gsems[1])
    for j in range(NCHUNK):
        b = j % 3
        wait_dma(bufs[b], gsems[b])
        scale(bufs[b], j)
        pltpu.async_copy(bufs[b], shared.at[dst_v.at[j]], ssems[b],
                         add=True)
        if j + 2 < NCHUNK:
            nb = (j + 2) % 3
            if j >= 1:
                wait_dma(bufs[nb], ssems[nb])
            pltpu.async_copy(y1_hbm.at[src_v.at[j + 2]], bufs[nb],
                             gsems[nb])
    for j in range(max(0, NCHUNK - 3), NCHUNK):
        wait_dma(bufs[j % 3], ssems[j % 3])
    plsc.subcore_barrier()
    pltpu.sync_copy(shared.at[pl.ds(s * ROWS_PT, ROWS_PT)],
                    out_hbm.at[c, pl.ds(s * ROWS_PT, ROWS_PT)])


import functools


@functools.lru_cache(maxsize=None)
def _sc_calls():
    mesh = plsc.VectorSubcoreMesh(core_axis_name="c", subcore_axis_name="s",
                                  num_cores=NC, num_subcores=NS)
    cparams = pltpu.CompilerParams(use_tc_tiling_on_sc=False)
    sc1 = pl.kernel(
        _sc1_body,
        out_type=jax.ShapeDtypeStruct((NC, NP, W1AUG), jnp.float32),
        mesh=mesh,
        compiler_params=cparams,
        scratch_types=[
            pltpu.VMEM((NCHUNK, CH), jnp.int32),
            pltpu.VMEM((NCHUNK, CH), jnp.int32),
            pltpu.VMEM((EPW,), jnp.float32),
            pltpu.VMEM((CH, W1AUG), jnp.float32),
            pltpu.VMEM((CH, W1AUG), jnp.float32),
            pltpu.VMEM((CH, W1AUG), jnp.float32),
            pltpu.VMEM_SHARED((NP, W1AUG), jnp.float32),
            pltpu.SemaphoreType.DMA,
            pltpu.SemaphoreType.DMA,
            pltpu.SemaphoreType.DMA,
            pltpu.SemaphoreType.DMA,
            pltpu.SemaphoreType.DMA,
            pltpu.SemaphoreType.DMA,
        ],
    )
    sc2 = pl.kernel(
        _sc2_body,
        out_type=jax.ShapeDtypeStruct((NC, NP, W2AUG), jnp.float32),
        mesh=mesh,
        compiler_params=cparams,
        scratch_types=[
            pltpu.VMEM((NCHUNK2, CH2), jnp.int32),
            pltpu.VMEM((NCHUNK2, CH2), jnp.int32),
            pltpu.VMEM((CH2, W2AUG), jnp.float32),
            pltpu.VMEM((CH2, W2AUG), jnp.float32),
            pltpu.VMEM((CH2, W2AUG), jnp.float32),
            pltpu.VMEM_SHARED((NP, W2AUG), jnp.float32),
            pltpu.SemaphoreType.DMA,
            pltpu.SemaphoreType.DMA,
            pltpu.SemaphoreType.DMA,
            pltpu.SemaphoreType.DMA,
            pltpu.SemaphoreType.DMA,
            pltpu.SemaphoreType.DMA,
        ],
    )
    return sc1, sc2


def _sc2_body(y2_hbm, src_hbm, dst_hbm, out_hbm,
              src_v, dst_v, rows_a, rows_b, rows_c, shared,
              gsem_a, gsem_b, gsem_c, ssem_a, ssem_b, ssem_c):
    c = lax.axis_index("c")
    s = lax.axis_index("s")
    wid = s * NC + c
    rows_a[pl.ds(0, ROWS_PT), :] = jnp.zeros((ROWS_PT, W2AUG), jnp.float32)
    pltpu.sync_copy(rows_a.at[pl.ds(0, ROWS_PT)],
                    shared.at[pl.ds(s * ROWS_PT, ROWS_PT)])
    pltpu.sync_copy(src_hbm.at[wid], src_v)
    pltpu.sync_copy(dst_hbm.at[wid], dst_v)
    plsc.subcore_barrier()

    bufs = (rows_a, rows_b, rows_c)
    gsems = (gsem_a, gsem_b, gsem_c)
    ssems = (ssem_a, ssem_b, ssem_c)

    def wait_dma(buf, sem):
        pltpu.make_async_copy(y2_hbm.at[pl.ds(0, CH2)], buf, sem).wait()

    pltpu.async_copy(y2_hbm.at[src_v.at[0]], bufs[0], gsems[0])
    pltpu.async_copy(y2_hbm.at[src_v.at[1]], bufs[1], gsems[1])
    for j in range(NCHUNK2):
        b = j % 3
        wait_dma(bufs[b], gsems[b])
        pltpu.async_copy(bufs[b], shared.at[dst_v.at[j]], ssems[b],
                         add=True)
        if j + 2 < NCHUNK2:
            nb = (j + 2) % 3
            if j >= 1:
                wait_dma(bufs[nb], ssems[nb])
            pltpu.async_copy(y2_hbm.at[src_v.at[j + 2]], bufs[nb],
                             gsems[nb])
    for j in range(max(0, NCHUNK2 - 3), NCHUNK2):
        wait_dma(bufs[j % 3], ssems[j % 3])
    plsc.subcore_barrier()
    pltpu.sync_copy(shared.at[pl.ds(s * ROWS_PT, ROWS_PT)],
                    out_hbm.at[c, pl.ds(s * ROWS_PT, ROWS_PT)])


def kernel(x, edge_index, weight, W1_rel, b1_rel, W1_root, b1_root,
           W2_rel, b2_rel, W2_root, b2_root):
    _sc1, _sc2 = _sc_calls()
    src = edge_index[0].reshape(NW, NCHUNK, CH)
    dst = edge_index[1].reshape(NW, NCHUNK, CH)
    src2 = edge_index[0].reshape(NW, NCHUNK2, CH2)
    dst2 = edge_index[1].reshape(NW, NCHUNK2, CH2)
    w2 = weight.reshape(NW, EPW)
    y1aug, z1 = _tc_a(x, W1_rel, W1_root,
                      b1_rel.reshape(1, H), b1_root.reshape(1, H))
    agg1 = _sc1(y1aug, src, dst, w2)
    y2aug, aux = _tc_b(agg1, z1, W2_rel, W2_root,
                       b2_rel.reshape(1, C), b2_root.reshape(1, C))
    agg2 = _sc2(y2aug, src2, dst2)
    return _tc_c(agg2, aux)


# speedup vs baseline: 22.7841x; 1.0941x over previous
import jax
import jax.numpy as jnp
from jax import lax
from jax.experimental import pallas as pl
from jax.experimental.pallas import tpu as pltpu
from jax.experimental.pallas import tpu_sc as plsc

N, E, D, H, C = 10000, 320000, 128, 32, 5
W1AUG = 40
W2AUG = 8
AUXW = 16
NC, NS = 2, 16
NW = NC * NS
EPW = E // NW
CH = 400
NCHUNK = EPW // CH
CH2 = 2000
NCHUNK2 = EPW // CH2
NP = 10240
ROWS_PT = NP // NS
BLK = 2000


def _tc_a_body(x_ref, wrel_ref, wroot_ref, brel_ref, broot_ref, y1_ref, z1_ref):
    x = x_ref[...]
    y = lax.dot_general(x, wrel_ref[...], (((1,), (1,)), ((), ())),
                        preferred_element_type=jnp.float32)
    ones = jnp.ones((BLK, 1), jnp.float32)
    pad = jnp.zeros((BLK, W1AUG - H - 1), jnp.float32)
    y1_ref[...] = jnp.concatenate([y, ones, pad], axis=1)
    z = lax.dot_general(x, wroot_ref[...], (((1,), (1,)), ((), ())),
                        preferred_element_type=jnp.float32)
    z1_ref[...] = z + brel_ref[...] + broot_ref[...]


_tc_a = pl.pallas_call(
    _tc_a_body,
    grid=(N // BLK,),
    in_specs=[
        pl.BlockSpec((BLK, D), lambda i: (i, 0)),
        pl.BlockSpec((H, D), lambda i: (0, 0)),
        pl.BlockSpec((H, D), lambda i: (0, 0)),
        pl.BlockSpec((1, H), lambda i: (0, 0)),
        pl.BlockSpec((1, H), lambda i: (0, 0)),
    ],
    out_specs=(pl.BlockSpec((BLK, W1AUG), lambda i: (i, 0)),
               pl.BlockSpec((BLK, H), lambda i: (i, 0))),
    out_shape=(jax.ShapeDtypeStruct((N, W1AUG), jnp.float32),
               jax.ShapeDtypeStruct((N, H), jnp.float32)),
)


def _tc_b_body(aggp_ref, z1_ref, wrel_ref, wroot_ref, brel_ref, broot_ref,
               y2_ref, aux_ref):
    a = aggp_ref[0] + aggp_ref[1]
    cnt = a[:, H:H + 1]
    out1 = a[:, :H] / jnp.maximum(cnt, 1.0) + z1_ref[...]
    nrm = jnp.sqrt(jnp.sum(out1 * out1, axis=1, keepdims=True))
    h = jnp.maximum(out1 / jnp.maximum(nrm, 1e-12), 0.0)
    y2 = lax.dot_general(h, wrel_ref[...], (((1,), (1,)), ((), ())),
                         preferred_element_type=jnp.float32)
    y2_ref[...] = jnp.concatenate(
        [y2, jnp.zeros((BLK, W2AUG - C), jnp.float32)], axis=1)
    z2 = lax.dot_general(h, wroot_ref[...], (((1,), (1,)), ((), ())),
                         preferred_element_type=jnp.float32)
    z2 = z2 + brel_ref[...] + broot_ref[...]
    aux_ref[...] = jnp.concatenate(
        [z2, cnt, jnp.zeros((BLK, AUXW - C - 1), jnp.float32)], axis=1)


_tc_b = pl.pallas_call(
    _tc_b_body,
    grid=(N // BLK,),
    in_specs=[
        pl.BlockSpec((NC, BLK, W1AUG), lambda i: (0, i, 0)),
        pl.BlockSpec((BLK, H), lambda i: (i, 0)),
        pl.BlockSpec((C, H), lambda i: (0, 0)),
        pl.BlockSpec((C, H), lambda i: (0, 0)),
        pl.BlockSpec((1, C), lambda i: (0, 0)),
        pl.BlockSpec((1, C), lambda i: (0, 0)),
    ],
    out_specs=(pl.BlockSpec((BLK, W2AUG), lambda i: (i, 0)),
               pl.BlockSpec((BLK, AUXW), lambda i: (i, 0))),
    out_shape=(jax.ShapeDtypeStruct((N, W2AUG), jnp.float32),
               jax.ShapeDtypeStruct((N, AUXW), jnp.float32)),
)


def _tc_c_body(agg2p_ref, aux_ref, out_ref):
    a = agg2p_ref[0] + agg2p_ref[1]
    aux = aux_ref[...]
    cnt = aux[:, C:C + 1]
    o = a[:, :C] / jnp.maximum(cnt, 1.0) + aux[:, :C]
    nrm = jnp.sqrt(jnp.sum(o * o, axis=1, keepdims=True))
    o = o / jnp.maximum(nrm, 1e-12)
    m = jnp.max(o, axis=1, keepdims=True)
    ls = (o - m) - jnp.log(jnp.sum(jnp.exp(o - m), axis=1, keepdims=True))
    out_ref[...] = ls


_tc_c = pl.pallas_call(
    _tc_c_body,
    grid=(N // BLK,),
    in_specs=[
        pl.BlockSpec((NC, BLK, W2AUG), lambda i: (0, i, 0)),
        pl.BlockSpec((BLK, AUXW), lambda i: (i, 0)),
    ],
    out_specs=pl.BlockSpec((BLK, C), lambda i: (i, 0)),
    out_shape=jax.ShapeDtypeStruct((N, C), jnp.float32),
)


def _sc1_body(y1_hbm, eidx_hbm, w_hbm, out_hbm,
              src_v, dst_v, w_v, rows_a, rows_b, rows_c, shared,
              gsem_a, gsem_b, gsem_c, ssem_a, ssem_b, ssem_c, isem):
    c = lax.axis_index("c")
    s = lax.axis_index("s")
    wid = s * NC + c
    base = wid * EPW
    for j in range(NCHUNK):
        pltpu.async_copy(eidx_hbm.at[0, pl.ds(base + j * CH, CH)],
                         src_v.at[j], isem)
        pltpu.async_copy(eidx_hbm.at[1, pl.ds(base + j * CH, CH)],
                         dst_v.at[j], isem)
    pltpu.sync_copy(w_hbm.at[pl.ds(base, EPW)], w_v)
    rows_a[...] = jnp.zeros((CH, W1AUG), jnp.float32)
    pltpu.sync_copy(rows_a, shared.at[pl.ds(s * ROWS_PT, CH)])
    pltpu.sync_copy(rows_a.at[pl.ds(0, ROWS_PT - CH)],
                    shared.at[pl.ds(s * ROWS_PT + CH, ROWS_PT - CH)])
    for j in range(2 * NCHUNK):
        pltpu.make_async_copy(eidx_hbm.at[0, pl.ds(0, CH)],
                              src_v.at[0], isem).wait()
    plsc.subcore_barrier()

    gdn = lax.GatherDimensionNumbers(
        offset_dims=(), collapsed_slice_dims=(0,), start_index_map=(0,))
    bufs = (rows_a, rows_b, rows_c)
    gsems = (gsem_a, gsem_b, gsem_c)
    ssems = (ssem_a, ssem_b, ssem_c)

    def scale(rows_v, j):
        def grp_body(g, carry2):
            wv = w_v[pl.ds(j * CH + g * 16, 16)]
            for l in range(16):
                wb = lax.gather(wv, jnp.full((16, 1), l, jnp.int32),
                                dimension_numbers=gdn, slice_sizes=(1,),
                                mode=lax.GatherScatterMode.PROMISE_IN_BOUNDS)
                e = g * 16 + l
                for cb in range(H // 16):
                    sl = (e, pl.ds(cb * 16, 16))
                    rows_v[sl] = rows_v[sl] * wb
            return carry2

        lax.fori_loop(0, CH // 16, grp_body, 0)

    def wait_dma(buf, sem):
        pltpu.make_async_copy(y1_hbm.at[pl.ds(0, CH)], buf, sem).wait()

    pltpu.async_copy(y1_hbm.at[src_v.at[0]], bufs[0], gsems[0])
    pltpu.async_copy(y1_hbm.at[src_v.at[1]], bufs[1], gsems[1])
    for j in range(NCHUNK):
        b = j % 3
        wait_dma(bufs[b], gsems[b])
        scale(bufs[b], j)
        pltpu.async_copy(bufs[b], shared.at[dst_v.at[j]], ssems[b],
                         add=True)
        if j + 2 < NCHUNK:
            nb = (j + 2) % 3
            if j >= 1:
                wait_dma(bufs[nb], ssems[nb])
            pltpu.async_copy(y1_hbm.at[src_v.at[j + 2]], bufs[nb],
                             gsems[nb])
    for j in range(max(0, NCHUNK - 3), NCHUNK):
        wait_dma(bufs[j % 3], ssems[j % 3])
    plsc.subcore_barrier()
    pltpu.sync_copy(shared.at[pl.ds(s * ROWS_PT, ROWS_PT)],
                    out_hbm.at[c, pl.ds(s * ROWS_PT, ROWS_PT)])


import functools


@functools.lru_cache(maxsize=None)
def _sc_calls():
    mesh = plsc.VectorSubcoreMesh(core_axis_name="c", subcore_axis_name="s",
                                  num_cores=NC, num_subcores=NS)
    cparams = pltpu.CompilerParams(use_tc_tiling_on_sc=False)
    sc1 = pl.kernel(
        _sc1_body,
        out_type=jax.ShapeDtypeStruct((NC, NP, W1AUG), jnp.float32),
        mesh=mesh,
        compiler_params=cparams,
        scratch_types=[
            pltpu.VMEM((NCHUNK, CH), jnp.int32),
            pltpu.VMEM((NCHUNK, CH), jnp.int32),
            pltpu.VMEM((EPW,), jnp.float32),
            pltpu.VMEM((CH, W1AUG), jnp.float32),
            pltpu.VMEM((CH, W1AUG), jnp.float32),
            pltpu.VMEM((CH, W1AUG), jnp.float32),
            pltpu.VMEM_SHARED((NP, W1AUG), jnp.float32),
            pltpu.SemaphoreType.DMA,
            pltpu.SemaphoreType.DMA,
            pltpu.SemaphoreType.DMA,
            pltpu.SemaphoreType.DMA,
            pltpu.SemaphoreType.DMA,
            pltpu.SemaphoreType.DMA,
            pltpu.SemaphoreType.DMA,
        ],
    )
    sc2 = pl.kernel(
        _sc2_body,
        out_type=jax.ShapeDtypeStruct((NC, NP, W2AUG), jnp.float32),
        mesh=mesh,
        compiler_params=cparams,
        scratch_types=[
            pltpu.VMEM((NCHUNK2, CH2), jnp.int32),
            pltpu.VMEM((NCHUNK2, CH2), jnp.int32),
            pltpu.VMEM((CH2, W2AUG), jnp.float32),
            pltpu.VMEM((CH2, W2AUG), jnp.float32),
            pltpu.VMEM((CH2, W2AUG), jnp.float32),
            pltpu.VMEM_SHARED((NP, W2AUG), jnp.float32),
            pltpu.SemaphoreType.DMA,
            pltpu.SemaphoreType.DMA,
            pltpu.SemaphoreType.DMA,
            pltpu.SemaphoreType.DMA,
            pltpu.SemaphoreType.DMA,
            pltpu.SemaphoreType.DMA,
            pltpu.SemaphoreType.DMA,
        ],
    )
    return sc1, sc2


def _sc2_body(y2_hbm, eidx_hbm, out_hbm,
              src_v, dst_v, rows_a, rows_b, rows_c, shared,
              gsem_a, gsem_b, gsem_c, ssem_a, ssem_b, ssem_c, isem):
    c = lax.axis_index("c")
    s = lax.axis_index("s")
    wid = s * NC + c
    base = wid * EPW
    for j in range(NCHUNK2):
        pltpu.async_copy(eidx_hbm.at[0, pl.ds(base + j * CH2, CH2)],
                         src_v.at[j], isem)
        pltpu.async_copy(eidx_hbm.at[1, pl.ds(base + j * CH2, CH2)],
                         dst_v.at[j], isem)
    rows_a[pl.ds(0, ROWS_PT), :] = jnp.zeros((ROWS_PT, W2AUG), jnp.float32)
    pltpu.sync_copy(rows_a.at[pl.ds(0, ROWS_PT)],
                    shared.at[pl.ds(s * ROWS_PT, ROWS_PT)])
    for j in range(2 * NCHUNK2):
        pltpu.make_async_copy(eidx_hbm.at[0, pl.ds(0, CH2)],
                              src_v.at[0], isem).wait()
    plsc.subcore_barrier()

    bufs = (rows_a, rows_b, rows_c)
    gsems = (gsem_a, gsem_b, gsem_c)
    ssems = (ssem_a, ssem_b, ssem_c)

    def wait_dma(buf, sem):
        pltpu.make_async_copy(y2_hbm.at[pl.ds(0, CH2)], buf, sem).wait()

    pltpu.async_copy(y2_hbm.at[src_v.at[0]], bufs[0], gsems[0])
    pltpu.async_copy(y2_hbm.at[src_v.at[1]], bufs[1], gsems[1])
    for j in range(NCHUNK2):
        b = j % 3
        wait_dma(bufs[b], gsems[b])
        pltpu.async_copy(bufs[b], shared.at[dst_v.at[j]], ssems[b],
                         add=True)
        if j + 2 < NCHUNK2:
            nb = (j + 2) % 3
            if j >= 1:
                wait_dma(bufs[nb], ssems[nb])
            pltpu.async_copy(y2_hbm.at[src_v.at[j + 2]], bufs[nb],
                             gsems[nb])
    for j in range(max(0, NCHUNK2 - 3), NCHUNK2):
        wait_dma(bufs[j % 3], ssems[j % 3])
    plsc.subcore_barrier()
    pltpu.sync_copy(shared.at[pl.ds(s * ROWS_PT, ROWS_PT)],
                    out_hbm.at[c, pl.ds(s * ROWS_PT, ROWS_PT)])


def kernel(x, edge_index, weight, W1_rel, b1_rel, W1_root, b1_root,
           W2_rel, b2_rel, W2_root, b2_root):
    _sc1, _sc2 = _sc_calls()
    y1aug, z1 = _tc_a(x, W1_rel, W1_root,
                      b1_rel.reshape(1, H), b1_root.reshape(1, H))
    agg1 = _sc1(y1aug, edge_index, weight)
    y2aug, aux = _tc_b(agg1, z1, W2_rel, W2_root,
                       b2_rel.reshape(1, C), b2_root.reshape(1, C))
    agg2 = _sc2(y2aug, edge_index)
    return _tc_c(agg2, aux)
